# baseline (speedup 1.0000x reference)
import jax
import jax.numpy as jnp
import numpy as np
from jax import lax
from jax.experimental import pallas as pl
from jax.experimental.pallas import tpu as pltpu

_N_ANCHORS = 3
_N_CLS = 80
_N_ATTR = _N_CLS + 5
_ANCHORS_WH = np.array([[116.0, 90.0], [156.0, 198.0], [373.0, 326.0]],
                       dtype=np.float32)


def _yolo_pane_kernel(params_ref, x_ref, out_ref):
    g = params_ref[7]
    stride = params_ref[0]
    a = pl.program_id(1)
    aw = jnp.where(a == 0, params_ref[1],
                   jnp.where(a == 1, params_ref[3], params_ref[5]))
    ah = jnp.where(a == 0, params_ref[2],
                   jnp.where(a == 1, params_ref[4], params_ref[6]))

    v = x_ref[0, 0]
    sig = jax.nn.sigmoid(v)

    h = v[0:8]
    shp = h.shape
    row = lax.broadcasted_iota(jnp.int32, shp, 0)
    colf = lax.broadcasted_iota(jnp.int32, shp, 1).astype(jnp.float32)
    gy = jnp.floor(colf / g)
    gx = colf - gy * g
    sig_h = sig[0:8]
    ex = jnp.minimum(jnp.exp(h), 1000.0) * jnp.where(row == 2, aw, ah)
    box = jnp.where(row < 2, (sig_h + jnp.where(row == 0, gx, gy)) * stride,
                    ex)
    head = jnp.where(row < 4, box, sig_h)
    res = jnp.concatenate([head, sig[8:]], axis=0)

    n = res.shape[0]
    eye = (lax.broadcasted_iota(jnp.int32, (n, n), 0)
           == lax.broadcasted_iota(jnp.int32, (n, n), 1)).astype(jnp.float32)
    out_ref[0, 0] = lax.dot_general(
        res, eye, (((0,), (0,)), ((), ())),
        preferred_element_type=jnp.float32)


def kernel(x, img_size):
    B = x.shape[0]
    G = x.shape[2]
    GG = G * G
    nA = _N_ANCHORS

    stride = jnp.float32(img_size) / jnp.float32(G)
    anch = jnp.asarray(_ANCHORS_WH)
    anch_eff = (anch / stride) * stride
    params = jnp.concatenate([
        stride[None], anch_eff.reshape(-1), jnp.float32(G)[None]
    ]).astype(jnp.float32)

    x4 = x.reshape(B, nA, _N_ATTR, GG)

    out4 = pl.pallas_call(
        _yolo_pane_kernel,
        grid=(B, nA),
        in_specs=[
            pl.BlockSpec(memory_space=pltpu.SMEM),
            pl.BlockSpec((1, 1, _N_ATTR, GG), lambda b, a: (b, a, 0, 0)),
        ],
        out_specs=pl.BlockSpec((1, 1, GG, _N_ATTR), lambda b, a: (b, a, 0, 0)),
        out_shape=jax.ShapeDtypeStruct((B, nA, GG, _N_ATTR), jnp.float32),
    )(params, x4)
    return out4.reshape(B, nA * GG, _N_ATTR)


# speedup vs baseline: 1.6452x; 1.6452x over previous
import jax
import jax.numpy as jnp
import numpy as np
from jax import lax
from jax.experimental import pallas as pl
from jax.experimental.pallas import tpu as pltpu

_N_ANCHORS = 3
_N_CLS = 80
_N_ATTR = _N_CLS + 5
_ANCHORS_WH = np.array([[116.0, 90.0], [156.0, 198.0], [373.0, 326.0]],
                       dtype=np.float32)


def _yolo_pane_kernel(params_ref, x_ref, out_ref):
    stride = params_ref[0]
    a = pl.program_id(1)
    aw = jnp.where(a == 0, params_ref[1],
                   jnp.where(a == 1, params_ref[3], params_ref[5]))
    ah = jnp.where(a == 0, params_ref[2],
                   jnp.where(a == 1, params_ref[4], params_ref[6]))

    v = x_ref[0]
    gg = v.shape[1] * v.shape[2]
    sig = jax.nn.sigmoid(v)

    h = v[0:8]
    shp = h.shape
    row = lax.broadcasted_iota(jnp.int32, shp, 0)
    gy = lax.broadcasted_iota(jnp.int32, shp, 1).astype(jnp.float32)
    gx = lax.broadcasted_iota(jnp.int32, shp, 2).astype(jnp.float32)
    sig_h = sig[0:8]
    ex = jnp.minimum(jnp.exp(h), 1000.0) * jnp.where(row == 2, aw, ah)
    box = jnp.where(row < 2, (sig_h + jnp.where(row == 0, gx, gy)) * stride,
                    ex)
    head = jnp.where(row < 4, box, sig_h)
    res = jnp.concatenate([head, sig[8:]], axis=0).reshape(_N_ATTR, gg)

    eye = (lax.broadcasted_iota(jnp.int32, (_N_ATTR, _N_ATTR), 0)
           == lax.broadcasted_iota(jnp.int32, (_N_ATTR, _N_ATTR), 1)
           ).astype(jnp.float32)
    out_ref[0] = lax.dot_general(
        res, eye, (((0,), (0,)), ((), ())),
        preferred_element_type=jnp.float32)


def kernel(x, img_size):
    B = x.shape[0]
    G = x.shape[2]
    GG = G * G
    nA = _N_ANCHORS

    stride = jnp.float32(img_size) / jnp.float32(G)
    anch = jnp.asarray(_ANCHORS_WH)
    anch_eff = (anch / stride) * stride
    params = jnp.concatenate([
        stride[None], anch_eff.reshape(-1)
    ]).astype(jnp.float32)

    return pl.pallas_call(
        _yolo_pane_kernel,
        grid=(B, nA),
        in_specs=[
            pl.BlockSpec(memory_space=pltpu.SMEM),
            pl.BlockSpec((1, _N_ATTR, G, G), lambda b, a: (b, a, 0, 0)),
        ],
        out_specs=pl.BlockSpec((1, GG, _N_ATTR), lambda b, a: (b, a, 0)),
        out_shape=jax.ShapeDtypeStruct((B, nA * GG, _N_ATTR), jnp.float32),
    )(params, x)
